# baseline (speedup 1.0000x reference)
import functools

import numpy as np
import jax
import jax.numpy as jnp
from jax import lax
from jax.experimental import pallas as pl
from jax.experimental.pallas import tpu as pltpu
from jax.experimental.pallas import tpu_sc as plsc

_N_ROWS = 16384
_SLICE = 4096
_D = 26 * 128

_NC, _NS = 2, 16
_NW = _NC * _NS
_B_PER_W = _SLICE // _NW
_CHUNK = 16
_NCHUNK = _B_PER_W // _CHUNK

_index_cache = None


def _perm_index() -> np.ndarray:
    global _index_cache
    if _index_cache is None:
        with jax.ensure_compile_time_eval():
            perm_key = jax.random.fold_in(jax.random.key(0), 1)
            perm = jax.random.permutation(perm_key, _N_ROWS)[:_SLICE]
        _index_cache = np.asarray(perm)
    return _index_cache


def _sc_gather(table, idx):
    mesh = plsc.VectorSubcoreMesh(core_axis_name="c", subcore_axis_name="s")

    @functools.partial(
        pl.kernel,
        mesh=mesh,
        out_type=jax.ShapeDtypeStruct((_SLICE, _D), jnp.float32),
        scratch_types=[
            pltpu.VMEM((_NCHUNK, _CHUNK), jnp.int32),
            pltpu.VMEM((_CHUNK, _D), jnp.float32),
            pltpu.VMEM((_CHUNK, _D), jnp.float32),
            pltpu.SemaphoreType.DMA,
            pltpu.SemaphoreType.DMA,
            pltpu.SemaphoreType.DMA,
            pltpu.SemaphoreType.DMA,
        ],
    )
    def k(table_hbm, idx_hbm, out_hbm, idx_v, buf0, buf1, gs0, gs1, os0, os1):
        wid = lax.axis_index("s") * _NC + lax.axis_index("c")
        base = wid * _B_PER_W
        pltpu.sync_copy(idx_hbm.at[wid], idx_v)
        bufs = (buf0, buf1)
        gsems = (gs0, gs1)
        osems = (os0, os1)
        gathers = [None, None]
        outs = [None, None]
        gathers[0] = pltpu.async_copy(table_hbm.at[idx_v.at[0]], buf0, gs0)
        for c in range(_NCHUNK):
            cur = c % 2
            nxt = (c + 1) % 2
            if c + 1 < _NCHUNK:
                if outs[nxt] is not None:
                    outs[nxt].wait()
                gathers[nxt] = pltpu.async_copy(
                    table_hbm.at[idx_v.at[c + 1]], bufs[nxt], gsems[nxt])
            gathers[cur].wait()
            outs[cur] = pltpu.async_copy(
                bufs[cur],
                out_hbm.at[pl.ds(base + c * _CHUNK, _CHUNK)],
                osems[cur])
        outs[0].wait()
        outs[1].wait()

    return k(table, idx)


def kernel(x):
    idx_np = _perm_index()
    table = x.reshape(_N_ROWS, _D)
    idx_dev = jnp.asarray(idx_np, dtype=jnp.int32).reshape(
        _NW, _NCHUNK, _CHUNK)
    out = _sc_gather(table, idx_dev)
    return (out.reshape(_SLICE, 26, 128), jnp.asarray(idx_np))


# speedup vs baseline: 1.6837x; 1.6837x over previous
import functools

import numpy as np
import jax
import jax.numpy as jnp
from jax import lax
from jax.experimental import pallas as pl
from jax.experimental.pallas import tpu as pltpu
from jax.experimental.pallas import tpu_sc as plsc

_N_ROWS = 16384
_SLICE = 4096
_SL, _LN = 26, 128

_NC, _NS = 2, 16
_NW = _NC * _NS
_B_PER_W = _SLICE // _NW
_CHUNK = 8
_NCHUNK = _B_PER_W // _CHUNK

_index_cache = None


def _perm_index() -> np.ndarray:
    global _index_cache
    if _index_cache is None:
        with jax.ensure_compile_time_eval():
            perm_key = jax.random.fold_in(jax.random.key(0), 1)
            perm = jax.random.permutation(perm_key, _N_ROWS)[:_SLICE]
        _index_cache = np.asarray(perm)
    return _index_cache


def _sc_gather(table, idx):
    mesh = plsc.VectorSubcoreMesh(core_axis_name="c", subcore_axis_name="s")

    @functools.partial(
        pl.kernel,
        mesh=mesh,
        out_type=jax.ShapeDtypeStruct((_SLICE, _SL, _LN), jnp.float32),
        scratch_types=[
            pltpu.VMEM((_NCHUNK, _CHUNK), jnp.int32),
            pltpu.VMEM((_CHUNK, _SL, _LN), jnp.float32),
            pltpu.VMEM((_CHUNK, _SL, _LN), jnp.float32),
            pltpu.SemaphoreType.DMA,
            pltpu.SemaphoreType.DMA,
            pltpu.SemaphoreType.DMA,
            pltpu.SemaphoreType.DMA,
        ],
    )
    def k(table_hbm, idx_hbm, out_hbm, idx_v, buf0, buf1, gs0, gs1, os0, os1):
        wid = lax.axis_index("s") * _NC + lax.axis_index("c")
        base = wid * _B_PER_W
        pltpu.sync_copy(idx_hbm.at[wid], idx_v)
        bufs = (buf0, buf1)
        gsems = (gs0, gs1)
        osems = (os0, os1)
        gathers = [None, None]
        outs = [None, None]
        gathers[0] = pltpu.async_copy(table_hbm.at[idx_v.at[0]], buf0, gs0)
        for c in range(_NCHUNK):
            cur = c % 2
            nxt = (c + 1) % 2
            if c + 1 < _NCHUNK:
                if outs[nxt] is not None:
                    outs[nxt].wait()
                gathers[nxt] = pltpu.async_copy(
                    table_hbm.at[idx_v.at[c + 1]], bufs[nxt], gsems[nxt])
            gathers[cur].wait()
            outs[cur] = pltpu.async_copy(
                bufs[cur],
                out_hbm.at[pl.ds(base + c * _CHUNK, _CHUNK)],
                osems[cur])
        outs[0].wait()
        outs[1].wait()

    return k(table, idx)


def kernel(x):
    idx_np = _perm_index()
    idx_dev = jnp.asarray(idx_np, dtype=jnp.int32).reshape(
        _NW, _NCHUNK, _CHUNK)
    out = _sc_gather(x, idx_dev)
    return (out, jnp.asarray(idx_np))


# speedup vs baseline: 1.6862x; 1.0015x over previous
import functools

import numpy as np
import jax
import jax.numpy as jnp
from jax import lax
from jax.experimental import pallas as pl
from jax.experimental.pallas import tpu as pltpu
from jax.experimental.pallas import tpu_sc as plsc

_N_ROWS = 16384
_SLICE = 4096
_SL, _LN = 26, 128

_NC, _NS = 2, 16
_NW = _NC * _NS
_B_PER_W = _SLICE // _NW
_CHUNK = 8
_NCHUNK = _B_PER_W // _CHUNK

_index_cache = None


def _perm_index() -> np.ndarray:
    global _index_cache
    if _index_cache is None:
        with jax.ensure_compile_time_eval():
            perm_key = jax.random.fold_in(jax.random.key(0), 1)
            perm = jax.random.permutation(perm_key, _N_ROWS)[:_SLICE]
        _index_cache = np.asarray(perm)
    return _index_cache


def _sc_gather(table, idx):
    mesh = plsc.VectorSubcoreMesh(core_axis_name="c", subcore_axis_name="s")

    @functools.partial(
        pl.kernel,
        mesh=mesh,
        compiler_params=pltpu.CompilerParams(use_tc_tiling_on_sc=True),
        out_type=jax.ShapeDtypeStruct((_SLICE, _SL, _LN), jnp.float32),
        scratch_types=[
            pltpu.VMEM((_NCHUNK, _CHUNK), jnp.int32),
            pltpu.VMEM((_CHUNK, _SL, _LN), jnp.float32),
            pltpu.VMEM((_CHUNK, _SL, _LN), jnp.float32),
            pltpu.SemaphoreType.DMA,
            pltpu.SemaphoreType.DMA,
            pltpu.SemaphoreType.DMA,
            pltpu.SemaphoreType.DMA,
        ],
    )
    def k(table_hbm, idx_hbm, out_hbm, idx_v, buf0, buf1, gs0, gs1, os0, os1):
        wid = lax.axis_index("s") * _NC + lax.axis_index("c")
        base = wid * _B_PER_W
        pltpu.sync_copy(idx_hbm.at[wid], idx_v)
        bufs = (buf0, buf1)
        gsems = (gs0, gs1)
        osems = (os0, os1)
        gathers = [None, None]
        outs = [None, None]
        gathers[0] = pltpu.async_copy(table_hbm.at[idx_v.at[0]], buf0, gs0)
        for c in range(_NCHUNK):
            cur = c % 2
            nxt = (c + 1) % 2
            if c + 1 < _NCHUNK:
                if outs[nxt] is not None:
                    outs[nxt].wait()
                gathers[nxt] = pltpu.async_copy(
                    table_hbm.at[idx_v.at[c + 1]], bufs[nxt], gsems[nxt])
            gathers[cur].wait()
            outs[cur] = pltpu.async_copy(
                bufs[cur],
                out_hbm.at[pl.ds(base + c * _CHUNK, _CHUNK)],
                osems[cur])
        outs[0].wait()
        outs[1].wait()

    return k(table, idx)


def kernel(x):
    idx_np = _perm_index()
    idx_dev = jnp.asarray(idx_np, dtype=jnp.int32).reshape(
        _NW, _NCHUNK, _CHUNK)
    out = _sc_gather(x, idx_dev)
    return (out, jnp.asarray(idx_np))


# speedup vs baseline: 7.4226x; 4.4020x over previous
import functools

import numpy as np
import jax
import jax.numpy as jnp
from jax import lax
from jax.experimental import pallas as pl
from jax.experimental.pallas import tpu as pltpu
from jax.experimental.pallas import tpu_sc as plsc

_N_ROWS = 16384
_SLICE = 4096
_SL, _LN = 26, 128

_FLAT_IN = _N_ROWS * _SL
_FLAT_OUT = _SLICE * _SL

_NC, _NS = 2, 16
_NW = _NC * _NS
_R_PER_W = _FLAT_OUT // _NW
_CHUNK = 128
_NCHUNK = _R_PER_W // _CHUNK
_PAIRS = _NCHUNK // 2

_index_cache = None


def _perm_index() -> np.ndarray:
    global _index_cache
    if _index_cache is None:
        with jax.ensure_compile_time_eval():
            perm_key = jax.random.fold_in(jax.random.key(0), 1)
            perm = jax.random.permutation(perm_key, _N_ROWS)[:_SLICE]
        _index_cache = np.asarray(perm)
    return _index_cache


def _sc_gather(table, idx):
    mesh = plsc.VectorSubcoreMesh(core_axis_name="c", subcore_axis_name="s")

    @functools.partial(
        pl.kernel,
        mesh=mesh,
        out_type=jax.ShapeDtypeStruct((_FLAT_OUT, _LN), jnp.float32),
        scratch_types=[
            pltpu.VMEM((_NCHUNK, _CHUNK), jnp.int32),
            pltpu.VMEM((_CHUNK, _LN), jnp.float32),
            pltpu.VMEM((_CHUNK, _LN), jnp.float32),
            pltpu.SemaphoreType.DMA,
            pltpu.SemaphoreType.DMA,
            pltpu.SemaphoreType.DMA,
            pltpu.SemaphoreType.DMA,
        ],
    )
    def k(table_hbm, idx_hbm, out_hbm, idx_v, buf0, buf1, gs0, gs1, os0, os1):
        wid = lax.axis_index("s") * _NC + lax.axis_index("c")
        base = wid * _R_PER_W
        pltpu.sync_copy(idx_hbm.at[wid], idx_v)

        def step(g, carry):
            a = 2 * g
            b = a + 1
            ga = pltpu.async_copy(table_hbm.at[idx_v.at[a]], buf0, gs0)
            gb = pltpu.async_copy(table_hbm.at[idx_v.at[b]], buf1, gs1)
            ga.wait()
            oa = pltpu.async_copy(
                buf0, out_hbm.at[pl.ds(base + a * _CHUNK, _CHUNK)], os0)
            gb.wait()
            ob = pltpu.async_copy(
                buf1, out_hbm.at[pl.ds(base + b * _CHUNK, _CHUNK)], os1)
            oa.wait()
            ob.wait()
            return carry

        lax.fori_loop(0, _PAIRS, step, 0)

    return k(table, idx)


def kernel(x):
    idx_np = _perm_index()
    flat_idx = (np.arange(_SL, dtype=np.int64)[:, None] * _N_ROWS
                + idx_np[None, :].astype(np.int64)).reshape(-1)
    idx_dev = jnp.asarray(flat_idx, dtype=jnp.int32).reshape(
        _NW, _NCHUNK, _CHUNK)
    table = jnp.transpose(x, (1, 0, 2)).reshape(_FLAT_IN, _LN)
    out_flat = _sc_gather(table, idx_dev)
    out = jnp.transpose(out_flat.reshape(_SL, _SLICE, _LN), (1, 0, 2))
    return (out, jnp.asarray(idx_np))


# speedup vs baseline: 7.8037x; 1.0513x over previous
import functools

import numpy as np
import jax
import jax.numpy as jnp
from jax import lax
from jax.experimental import pallas as pl
from jax.experimental.pallas import tpu as pltpu
from jax.experimental.pallas import tpu_sc as plsc

_N_ROWS = 16384
_SLICE = 4096
_SL, _LN = 26, 128

_FLAT_IN = _N_ROWS * _SL
_FLAT_OUT = _SLICE * _SL

_NC, _NS = 2, 16
_NW = _NC * _NS
_R_PER_W = _FLAT_OUT // _NW
_CHUNK = 104
_NCHUNK = _R_PER_W // _CHUNK
_QUADS = _NCHUNK // 4

_index_cache = None


def _perm_index() -> np.ndarray:
    global _index_cache
    if _index_cache is None:
        with jax.ensure_compile_time_eval():
            perm_key = jax.random.fold_in(jax.random.key(0), 1)
            perm = jax.random.permutation(perm_key, _N_ROWS)[:_SLICE]
        _index_cache = np.asarray(perm)
    return _index_cache


def _sc_gather(table, idx):
    mesh = plsc.VectorSubcoreMesh(core_axis_name="c", subcore_axis_name="s")

    @functools.partial(
        pl.kernel,
        mesh=mesh,
        out_type=jax.ShapeDtypeStruct((_FLAT_OUT, _LN), jnp.float32),
        scratch_types=[
            pltpu.VMEM((_NCHUNK, _CHUNK), jnp.int32),
            pltpu.VMEM((_CHUNK, _LN), jnp.float32),
            pltpu.VMEM((_CHUNK, _LN), jnp.float32),
            pltpu.VMEM((_CHUNK, _LN), jnp.float32),
            pltpu.VMEM((_CHUNK, _LN), jnp.float32),
            pltpu.SemaphoreType.DMA,
            pltpu.SemaphoreType.DMA,
            pltpu.SemaphoreType.DMA,
            pltpu.SemaphoreType.DMA,
            pltpu.SemaphoreType.DMA,
            pltpu.SemaphoreType.DMA,
            pltpu.SemaphoreType.DMA,
            pltpu.SemaphoreType.DMA,
        ],
    )
    def k(table_hbm, idx_hbm, out_hbm, idx_v,
          bufa0, bufa1, bufb0, bufb1,
          ga0, ga1, gb0, gb1, oa0, oa1, ob0, ob1):
        wid = lax.axis_index("s") * _NC + lax.axis_index("c")
        base = wid * _R_PER_W
        pltpu.sync_copy(idx_hbm.at[wid], idx_v)

        def gather(c, buf, sem):
            return pltpu.async_copy(table_hbm.at[idx_v.at[c]], buf, sem)

        def put(c, buf, sem):
            return pltpu.async_copy(
                buf, out_hbm.at[pl.ds(base + c * _CHUNK, _CHUNK)], sem)

        def step(g, carry):
            c0 = 4 * g
            g0 = gather(c0, bufa0, ga0)
            g1 = gather(c0 + 1, bufa1, ga1)
            g0.wait()
            o0 = put(c0, bufa0, oa0)
            g2 = gather(c0 + 2, bufb0, gb0)
            g1.wait()
            o1 = put(c0 + 1, bufa1, oa1)
            g3 = gather(c0 + 3, bufb1, gb1)
            g2.wait()
            o2 = put(c0 + 2, bufb0, ob0)
            g3.wait()
            o3 = put(c0 + 3, bufb1, ob1)
            o0.wait()
            o1.wait()
            o2.wait()
            o3.wait()
            return carry

        lax.fori_loop(0, _QUADS, step, 0)

    return k(table, idx)


def kernel(x):
    idx_np = _perm_index()
    flat_idx = (np.arange(_SL, dtype=np.int64)[:, None] * _N_ROWS
                + idx_np[None, :].astype(np.int64)).reshape(-1)
    idx_dev = jnp.asarray(flat_idx, dtype=jnp.int32).reshape(
        _NW, _NCHUNK, _CHUNK)
    table = jnp.transpose(x, (1, 0, 2)).reshape(_FLAT_IN, _LN)
    out_flat = _sc_gather(table, idx_dev)
    out = jnp.transpose(out_flat.reshape(_SL, _SLICE, _LN), (1, 0, 2))
    return (out, jnp.asarray(idx_np))


# speedup vs baseline: 8.1404x; 1.0431x over previous
import functools

import numpy as np
import jax
import jax.numpy as jnp
from jax import lax
from jax.experimental import pallas as pl
from jax.experimental.pallas import tpu as pltpu
from jax.experimental.pallas import tpu_sc as plsc

_N_ROWS = 16384
_SLICE = 4096
_SL, _LN = 26, 128

_FLAT_IN = _N_ROWS * _SL
_FLAT_OUT = _SLICE * _SL

_NC, _NS = 2, 16
_NW = _NC * _NS
_R_PER_W = _FLAT_OUT // _NW
_CHUNK = 104
_NCHUNK = _R_PER_W // _CHUNK
_QUADS = _NCHUNK // 4

_index_cache = None


def _perm_index() -> np.ndarray:
    global _index_cache
    if _index_cache is None:
        with jax.ensure_compile_time_eval():
            perm_key = jax.random.fold_in(jax.random.key(0), 1)
            perm = jax.random.permutation(perm_key, _N_ROWS)[:_SLICE]
        _index_cache = np.asarray(perm)
    return _index_cache


def _sc_gather(table, idx):
    mesh = plsc.VectorSubcoreMesh(core_axis_name="c", subcore_axis_name="s")

    @functools.partial(
        pl.kernel,
        mesh=mesh,
        out_type=jax.ShapeDtypeStruct((_FLAT_OUT, _LN), jnp.float32),
        scratch_types=[
            pltpu.VMEM((_NCHUNK, _CHUNK), jnp.int32),
            pltpu.VMEM((_CHUNK, _LN), jnp.float32),
            pltpu.VMEM((_CHUNK, _LN), jnp.float32),
            pltpu.VMEM((_CHUNK, _LN), jnp.float32),
            pltpu.VMEM((_CHUNK, _LN), jnp.float32),
            pltpu.SemaphoreType.DMA,
            pltpu.SemaphoreType.DMA,
            pltpu.SemaphoreType.DMA,
            pltpu.SemaphoreType.DMA,
            pltpu.SemaphoreType.DMA,
            pltpu.SemaphoreType.DMA,
            pltpu.SemaphoreType.DMA,
            pltpu.SemaphoreType.DMA,
        ],
    )
    def k(table_hbm, idx_hbm, out_hbm, idx_v,
          bufa0, bufa1, bufb0, bufb1,
          ga0, ga1, gb0, gb1, oa0, oa1, ob0, ob1):
        wid = lax.axis_index("s") * _NC + lax.axis_index("c")
        base = wid * _R_PER_W
        pltpu.sync_copy(idx_hbm.at[wid], idx_v)

        def gather(c, buf, sem):
            return pltpu.async_copy(table_hbm.at[idx_v.at[c]], buf, sem)

        def put(c, buf, sem):
            return pltpu.async_copy(
                buf, out_hbm.at[pl.ds(base + c * _CHUNK, _CHUNK)], sem)

        def drain(buf, sem):
            pltpu.make_async_copy(
                buf, out_hbm.at[pl.ds(base, _CHUNK)], sem).wait()

        def step(g, carry):
            c0 = 4 * g

            @pl.when(g > 0)
            def _():
                drain(bufa0, oa0)
                drain(bufa1, oa1)

            g0 = gather(c0, bufa0, ga0)
            g1 = gather(c0 + 1, bufa1, ga1)

            @pl.when(g > 0)
            def _():
                drain(bufb0, ob0)
                drain(bufb1, ob1)

            g0.wait()
            put(c0, bufa0, oa0)
            g2 = gather(c0 + 2, bufb0, gb0)
            g1.wait()
            put(c0 + 1, bufa1, oa1)
            g3 = gather(c0 + 3, bufb1, gb1)
            g2.wait()
            put(c0 + 2, bufb0, ob0)
            g3.wait()
            put(c0 + 3, bufb1, ob1)
            return carry

        lax.fori_loop(0, _QUADS, step, 0)
        drain(bufa0, oa0)
        drain(bufa1, oa1)
        drain(bufb0, ob0)
        drain(bufb1, ob1)

    return k(table, idx)


def kernel(x):
    idx_np = _perm_index()
    flat_idx = (np.arange(_SL, dtype=np.int64)[:, None] * _N_ROWS
                + idx_np[None, :].astype(np.int64)).reshape(-1)
    idx_dev = jnp.asarray(flat_idx, dtype=jnp.int32).reshape(
        _NW, _NCHUNK, _CHUNK)
    table = jnp.transpose(x, (1, 0, 2)).reshape(_FLAT_IN, _LN)
    out_flat = _sc_gather(table, idx_dev)
    out = jnp.transpose(out_flat.reshape(_SL, _SLICE, _LN), (1, 0, 2))
    return (out, jnp.asarray(idx_np))
